# baseline (speedup 1.0000x reference)
import functools

import jax
import jax.numpy as jnp
from jax import lax
from jax.experimental import pallas as pl
from jax.experimental.pallas import tpu as pltpu
from jax.experimental.pallas import tpu_sc as plsc

N = 35000
T = 35
D = 128
H = 32
HG = 64
OUT = 2
E = 1120000
G = N // T

NTILE = 16
NP = 36864
STRIPE = NP // NTILE
EW = 125
ROWS_PER_TILE = E // EW // NTILE
OBLK = 8
NOUT = ROWS_PER_TILE // OBLK

@functools.cache
def _sc_mesh():
    return plsc.VectorSubcoreMesh(core_axis_name="c", subcore_axis_name="s")


def _mm_bias_body(x_ref, w_ref, b_ref, o_ref):
    o_ref[0] = (jnp.dot(x_ref[0], w_ref[0], preferred_element_type=jnp.float32)
                + b_ref[0])


def _mm_bias(x, w, b, bm):
    s, m, k = w.shape[0], x.shape[1], x.shape[2]
    n = w.shape[2]
    return pl.pallas_call(
        _mm_bias_body,
        grid=(s, m // bm),
        in_specs=[
            pl.BlockSpec((1, bm, k), lambda i, j: (i // 2, j, 0)),
            pl.BlockSpec((1, k, n), lambda i, j: (i, 0, 0)),
            pl.BlockSpec((1, 1, n), lambda i, j: (i, 0, 0)),
        ],
        out_specs=pl.BlockSpec((1, bm, n), lambda i, j: (i, j, 0)),
        out_shape=jax.ShapeDtypeStruct((s, m, n), jnp.float32),
    )(x, w, b.reshape(s, 1, n))


TCH = 7
NCH = T // TCH


def _scan_body(x_ref, wih_ref, whh_ref, b_ref, o_ref, h_sc, c_sc):
    s = pl.program_id(0)
    j = pl.program_id(1)
    is_bwd = s % 2
    wih = wih_ref[0]
    whh = whh_ref[0]
    b = b_ref[0]

    @pl.when(j == 0)
    def _():
        h_sc[...] = jnp.zeros((G, H), jnp.float32)
        c_sc[...] = jnp.zeros((G, H), jnp.float32)

    din = x_ref.shape[3]
    xg = jnp.dot(x_ref[0].reshape(TCH * G, din), wih,
                 preferred_element_type=jnp.float32)

    def chain(order):
        for te in order:
            h = h_sc[...]
            g = (xg[te * G:(te + 1) * G]
                 + jnp.dot(h, whh, preferred_element_type=jnp.float32) + b)
            sg = jax.nn.sigmoid(g[:, 0:3 * H])
            tg = jnp.tanh(g[:, 3 * H:4 * H])
            c = sg[:, H:2 * H] * c_sc[...] + sg[:, 0:H] * tg
            h = sg[:, 2 * H:3 * H] * jnp.tanh(c)
            h_sc[...] = h
            c_sc[...] = c
            o_ref[0, te] = h

    pl.when(is_bwd == 0)(lambda: chain(range(TCH)))
    pl.when(is_bwd == 1)(lambda: chain(range(TCH - 1, -1, -1)))


def _chunk_map(i, j):
    return jnp.where(i % 2 == 1, NCH - 1 - j, j)


def _lstm_scan(x, wih, whh, b):
    s, din = wih.shape[0], wih.shape[1]
    return pl.pallas_call(
        _scan_body,
        grid=(s, NCH),
        in_specs=[
            pl.BlockSpec((1, TCH, G, din),
                         lambda i, j: (i // 2, _chunk_map(i, j), 0, 0)),
            pl.BlockSpec((1, din, 4 * H), lambda i, j: (i, 0, 0)),
            pl.BlockSpec((1, H, 4 * H), lambda i, j: (i, 0, 0)),
            pl.BlockSpec((1, 1, 4 * H), lambda i, j: (i, 0, 0)),
        ],
        out_specs=pl.BlockSpec((1, TCH, G, H),
                               lambda i, j: (i, _chunk_map(i, j), 0, 0)),
        out_shape=jax.ShapeDtypeStruct((s, T, G, H), jnp.float32),
        scratch_shapes=[
            pltpu.VMEM((G, H), jnp.float32),
            pltpu.VMEM((G, H), jnp.float32),
        ],
    )(x, wih, whh, b.reshape(s, 1, 4 * H))


def _pre_body(wf_ref, dego_ref, linwt_ref, linb_ref, cw_ref, l2g_ref, xa_ref,
              xb_ref):
    wf = wf_ref[0]
    s_out = lax.rsqrt(jnp.maximum(dego_ref[0], 1.0))
    l2g_ref[0] = jnp.maximum(
        jnp.dot(wf, linwt_ref[...], preferred_element_type=jnp.float32)
        + linb_ref[...], 0.0)
    xw = jnp.dot(wf * s_out, cw_ref[0], preferred_element_type=jnp.float32)
    xa_ref[0] = xw[:, 0:HG // 2]
    xb_ref[0] = xw[:, HG // 2:HG]


def _pre_stage(wf, dego, lin_WT, lin_b, cw1, bm):
    return pl.pallas_call(
        _pre_body,
        grid=(1, N // bm),
        in_specs=[
            pl.BlockSpec((1, bm, HG), lambda i, j: (i, j, 0)),
            pl.BlockSpec((1, bm, 1), lambda i, j: (i, j, 0)),
            pl.BlockSpec((HG, HG), lambda i, j: (0, 0)),
            pl.BlockSpec((HG,), lambda i, j: (0,)),
            pl.BlockSpec((1, HG, HG), lambda i, j: (i, 0, 0)),
        ],
        out_specs=[
            pl.BlockSpec((1, bm, HG), lambda i, j: (i, j, 0)),
            pl.BlockSpec((1, bm, HG // 2), lambda i, j: (i, j, 0)),
            pl.BlockSpec((1, bm, HG // 2), lambda i, j: (i, j, 0)),
        ],
        out_shape=[
            jax.ShapeDtypeStruct((1, N, HG), jnp.float32),
            jax.ShapeDtypeStruct((1, N, HG // 2), jnp.float32),
            jax.ShapeDtypeStruct((1, N, HG // 2), jnp.float32),
        ],
    )(wf, dego, lin_WT, lin_b, cw1)


def _mid_body(aa_ref, ab_ref, degi_ref, dego_ref, l2g_ref, cb_ref, cw_ref,
              xa_ref, xb_ref):
    s_in = lax.rsqrt(jnp.maximum(degi_ref[0], 1.0))
    s_out = lax.rsqrt(jnp.maximum(dego_ref[0], 1.0))
    cb = cb_ref[0, 0]
    l2g = l2g_ref[0]
    Hh = HG // 2
    h1a = (jnp.maximum(aa_ref[0] * s_in + cb[0:Hh], 0.0) * 0.6
           + 0.4 * l2g[:, 0:Hh]) * s_out
    h1b = (jnp.maximum(ab_ref[0] * s_in + cb[Hh:HG], 0.0) * 0.6
           + 0.4 * l2g[:, Hh:HG]) * s_out
    cw = cw_ref[0]
    xw = (jnp.dot(h1a, cw[0:Hh], preferred_element_type=jnp.float32)
          + jnp.dot(h1b, cw[Hh:HG], preferred_element_type=jnp.float32))
    xa_ref[0] = xw[:, 0:Hh]
    xb_ref[0] = xw[:, Hh:HG]


def _mid_stage(agg, degi, dego, l2g, cb1, cw2, bm):
    Hh = HG // 2
    return pl.pallas_call(
        _mid_body,
        grid=(1, N // bm),
        in_specs=[
            pl.BlockSpec((1, bm, Hh), lambda i, j: (2 * i, j, 0)),
            pl.BlockSpec((1, bm, Hh), lambda i, j: (2 * i + 1, j, 0)),
            pl.BlockSpec((1, bm, 1), lambda i, j: (i, j, 0)),
            pl.BlockSpec((1, bm, 1), lambda i, j: (i, j, 0)),
            pl.BlockSpec((1, bm, HG), lambda i, j: (i, j, 0)),
            pl.BlockSpec((1, 1, HG), lambda i, j: (i, 0, 0)),
            pl.BlockSpec((1, HG, HG), lambda i, j: (i, 0, 0)),
        ],
        out_specs=[
            pl.BlockSpec((1, bm, Hh), lambda i, j: (i, j, 0)),
            pl.BlockSpec((1, bm, Hh), lambda i, j: (i, j, 0)),
        ],
        out_shape=[
            jax.ShapeDtypeStruct((1, N, Hh), jnp.float32),
            jax.ShapeDtypeStruct((1, N, Hh), jnp.float32),
        ],
    )(agg, agg, degi, dego, l2g, cb1.reshape(1, 1, HG), cw2)


def _final_body(a1_ref, a2_ref, degi_ref, l2g_ref, cb_ref, predwt_ref,
                predb_ref, o_ref, *, mb):
    gb = mb // T
    r = lax.broadcasted_iota(jnp.int32, (gb, mb), 0)
    cdiv = lax.broadcasted_iota(jnp.int32, (gb, mb), 1) // T
    P = jnp.where(r == cdiv, 1.0 / T, 0.0).astype(jnp.float32)
    Hh = HG // 2
    reps = []
    for s, ag_ref in ((0, a1_ref), (1, a2_ref)):
        s_in = lax.rsqrt(jnp.maximum(degi_ref[s], 1.0))
        cb = cb_ref[s]
        l2g = l2g_ref[s]
        h2a = (jnp.maximum(ag_ref[0] * s_in + cb[0:Hh], 0.0) * 0.6
               + 0.4 * l2g[:, 0:Hh])
        h2b = (jnp.maximum(ag_ref[1] * s_in + cb[Hh:HG], 0.0) * 0.6
               + 0.4 * l2g[:, Hh:HG])
        ra = jnp.dot(P, h2a, preferred_element_type=jnp.float32)
        rb = jnp.dot(P, h2b, preferred_element_type=jnp.float32)
        reps.append((ra, rb))
    da = jnp.tanh(reps[0][0] - reps[1][0])
    db = jnp.tanh(reps[0][1] - reps[1][1])
    pw = predwt_ref[...]
    o_ref[...] = (jnp.dot(da, pw[0:Hh], preferred_element_type=jnp.float32)
                  + jnp.dot(db, pw[Hh:HG], preferred_element_type=jnp.float32)
                  + predb_ref[...])


def _final_stage(a1, a2, degi, l2g, cb2, pred_WT, pred_b, mb):
    Hh = HG // 2
    return pl.pallas_call(
        functools.partial(_final_body, mb=mb),
        grid=(N // mb,),
        in_specs=[
            pl.BlockSpec((2, mb, Hh), lambda j: (0, j, 0)),
            pl.BlockSpec((2, mb, Hh), lambda j: (0, j, 0)),
            pl.BlockSpec((2, mb, 1), lambda j: (0, j, 0)),
            pl.BlockSpec((2, mb, HG), lambda j: (0, j, 0)),
            pl.BlockSpec((2, HG), lambda j: (0, 0)),
            pl.BlockSpec((HG, OUT), lambda j: (0, 0)),
            pl.BlockSpec((OUT,), lambda j: (0,)),
        ],
        out_specs=pl.BlockSpec((mb // T, OUT), lambda j: (j, 0)),
        out_shape=jax.ShapeDtypeStruct((G, OUT), jnp.float32),
    )(a1, a2, degi, l2g, cb2, pred_WT, pred_b)


def _deg_body(e1_ref, e2_ref, deg_ref, hs_ref, hd_ref, idx_ref, ones_ref,
              zb_ref, dsem):
    c = lax.axis_index("c")
    t = lax.axis_index("s")

    def fill(i, _):
        zb_ref[pl.ds(i * 16, 16)] = jnp.zeros((16,), jnp.float32)
        return 0

    lax.fori_loop(0, STRIPE // 16, fill, 0)
    for i in range(8):
        ones_ref[pl.ds(i * 16, 16)] = jnp.full((16,), 1.0, jnp.float32)
    pltpu.sync_copy(zb_ref, hs_ref.at[pl.ds(t * STRIPE, STRIPE)])
    pltpu.sync_copy(zb_ref, hd_ref.at[pl.ds(t * STRIPE, STRIPE)])
    plsc.subcore_barrier()

    def run(e_ref, dsem):
        for a, hist in ((0, hs_ref), (1, hd_ref)):
            def outer(o, _):
                pltpu.sync_copy(e_ref.at[a, t, pl.ds(o * OBLK, OBLK)],
                                idx_ref)
                scats = [
                    pltpu.async_copy(ones_ref.at[pl.ds(0, EW)],
                                     hist.at[idx_ref.at[b]], dsem, add=True)
                    for b in range(OBLK)
                ]
                for s_ in scats:
                    s_.wait()
                return 0

            lax.fori_loop(0, NOUT, outer, 0)

    pl.when(c == 0)(lambda: run(e1_ref, dsem))
    pl.when(c == 1)(lambda: run(e2_ref, dsem))
    plsc.subcore_barrier()
    pltpu.sync_copy(hs_ref.at[pl.ds(t * STRIPE, STRIPE)],
                    deg_ref.at[2 * c, 0, pl.ds(t * STRIPE, STRIPE)])
    pltpu.sync_copy(hd_ref.at[pl.ds(t * STRIPE, STRIPE)],
                    deg_ref.at[2 * c + 1, 0, pl.ds(t * STRIPE, STRIPE)])


def _sc_degrees(e1, e2):
    return pl.kernel(
        _deg_body,
        mesh=_sc_mesh(),
        compiler_params=pltpu.CompilerParams(use_tc_tiling_on_sc=False),
        out_type=jax.ShapeDtypeStruct((4, 1, NP), jnp.float32),
        scratch_types=[
            pltpu.VMEM_SHARED((NP,), jnp.float32),
            pltpu.VMEM_SHARED((NP,), jnp.float32),
            pltpu.VMEM((OBLK, EW), jnp.int32),
            pltpu.VMEM((128,), jnp.float32),
            pltpu.VMEM((STRIPE,), jnp.float32),
            pltpu.SemaphoreType.DMA,
        ],
    )(e1, e2)


ZROWS = 144
AIBLK = 40
ANOUT = ROWS_PER_TILE // AIBLK
SUBW = 5


def _agg_body(xa_ref, xb_ref, e_ref, out_ref, acc_ref, isrc_ref, idst_ref,
              rows_ref, zb_ref, gsem, ssem0, ssem1):
    c = lax.axis_index("c")
    t = lax.axis_index("s")
    ssems = (ssem0, ssem1)

    def fill(i, _):
        zb_ref[i, 0:16] = jnp.zeros((16,), jnp.float32)
        zb_ref[i, 16:32] = jnp.zeros((16,), jnp.float32)
        return 0

    lax.fori_loop(0, ZROWS, fill, 0)

    def drain_set(s):
        for b in range(SUBW):
            pltpu.make_async_copy(rows_ref.at[s, b],
                                  acc_ref.at[idst_ref.at[0]],
                                  ssems[s]).wait()

    def run_phase(xw_ref):
        for z in range(STRIPE // ZROWS):
            pltpu.sync_copy(
                zb_ref, acc_ref.at[pl.ds(t * STRIPE + z * ZROWS, ZROWS)])
        plsc.subcore_barrier()

        def outer(oi, _):
            pltpu.sync_copy(e_ref.at[0, t, pl.ds(oi * AIBLK, AIBLK)],
                            isrc_ref)
            pltpu.sync_copy(e_ref.at[1, t, pl.ds(oi * AIBLK, AIBLK)],
                            idst_ref)

            def sub(kp, _):
                for s in range(2):
                    k = kp * 2 + s
                    pl.when(jnp.logical_or(oi > 0, kp > 0))(
                        lambda s=s: drain_set(s))
                    gs = [
                        pltpu.async_copy(
                            xw_ref.at[isrc_ref.at[k * SUBW + b]],
                            rows_ref.at[s, b], gsem)
                        for b in range(SUBW)
                    ]
                    for g_ in gs:
                        g_.wait()
                    for b in range(SUBW):
                        pltpu.async_copy(rows_ref.at[s, b],
                                         acc_ref.at[idst_ref.at[k * SUBW + b]],
                                         ssems[s], add=True)
                return 0

            lax.fori_loop(0, AIBLK // SUBW // 2, sub, 0)
            return 0

        lax.fori_loop(0, ANOUT, outer, 0)
        for s in range(2):
            drain_set(s)
        plsc.subcore_barrier()
        pltpu.sync_copy(acc_ref.at[pl.ds(t * STRIPE, STRIPE)],
                        out_ref.at[c, pl.ds(t * STRIPE, STRIPE)])

    pl.when(c == 0)(lambda: run_phase(xa_ref))
    pl.when(c == 1)(lambda: run_phase(xb_ref))


def _sc_aggregate(xa, xb, e):
    return pl.kernel(
        _agg_body,
        mesh=_sc_mesh(),
        compiler_params=pltpu.CompilerParams(use_tc_tiling_on_sc=False),
        out_type=jax.ShapeDtypeStruct((2, NP, HG // 2), jnp.float32),
        scratch_types=[
            pltpu.VMEM_SHARED((NP, HG // 2), jnp.float32),
            pltpu.VMEM((AIBLK, EW), jnp.int32),
            pltpu.VMEM((AIBLK, EW), jnp.int32),
            pltpu.VMEM((2, SUBW, EW, HG // 2), jnp.float32),
            pltpu.VMEM((ZROWS, HG // 2), jnp.float32),
            pltpu.SemaphoreType.DMA,
            pltpu.SemaphoreType.DMA,
            pltpu.SemaphoreType.DMA,
        ],
    )(xa, xb, e)


def kernel(x1, edge_index1, x2, edge_index2, rnn1, rnn2, lin_W, lin_b,
           c11_W, c11_b, c21_W, c21_b, c12_W, c12_b, c22_W, c22_b,
           pred_W, pred_b):
    f32 = jnp.float32

    def gate_perm(w):
        return jnp.concatenate(
            [w[..., 0:2 * H], w[..., 3 * H:4 * H], w[..., 2 * H:3 * H]], -1)

    def layer_w(rnns, layer):
        wt, bt, ut = [], [], []
        for rnn in rnns:
            for d in range(2):
                Wih, Whh, bih, bhh = rnn[layer][d]
                wt.append(gate_perm(Wih.T))
                bt.append(gate_perm(bih + bhh))
                ut.append(gate_perm(Whh.T))
        return jnp.stack(wt), jnp.stack(bt), jnp.stack(ut)

    w1t, b1, u1t = layer_w((rnn1, rnn2), 0)
    w2t, b2, u2t = layer_w((rnn1, rnn2), 1)

    e1 = edge_index1.reshape(2, NTILE, ROWS_PER_TILE, EW)
    e2 = edge_index2.reshape(2, NTILE, ROWS_PER_TILE, EW)

    degs = _sc_degrees(e1, e2)
    dego = degs[0::2, 0, :N, None]
    degi = degs[1::2, 0, :N, None]

    def bilstm(x, w1t_i, u1t_i, b1_i, w2t_i, u2t_i, b2_i):
        xt = x.reshape(G, T, D).transpose(1, 0, 2).reshape(1, T, G, D)
        hs1 = _lstm_scan(xt, w1t_i, u1t_i, b1_i)
        wf_t = jnp.concatenate([hs1[0], hs1[1]], -1)[None]
        hs2 = _lstm_scan(wf_t, w2t_i, u2t_i, b2_i)
        return (jnp.concatenate([hs2[0], hs2[1]], -1)
                .transpose(1, 0, 2).reshape(1, N, 2 * H))

    wf1 = bilstm(x1, w1t[0:2], u1t[0:2], b1[0:2], w2t[0:2], u2t[0:2], b2[0:2])
    wf2 = bilstm(x2, w1t[2:4], u1t[2:4], b1[2:4], w2t[2:4], u2t[2:4], b2[2:4])

    dego1, degi1 = degs[0, 0, :N, None][None], degs[1, 0, :N, None][None]
    dego2, degi2 = degs[2, 0, :N, None][None], degs[3, 0, :N, None][None]

    l2g1, xa1, xb1 = _pre_stage(wf1, dego1, lin_W.T, lin_b, c11_W[None],
                                bm=5000)
    agg1_g1 = _sc_aggregate(xa1[0], xb1[0], e1)
    l2g2, xa2, xb2 = _pre_stage(wf2, dego2, lin_W.T, lin_b, c12_W[None],
                                bm=5000)
    agg1_g2 = _sc_aggregate(xa2[0], xb2[0], e2)
    ya1, yb1 = _mid_stage(agg1_g1, degi1, dego1, l2g1, c11_b[None],
                          c21_W[None], bm=5000)
    agg2_g1 = _sc_aggregate(ya1[0], yb1[0], e1)
    ya2, yb2 = _mid_stage(agg1_g2, degi2, dego2, l2g2, c12_b[None],
                          c22_W[None], bm=5000)
    agg2_g2 = _sc_aggregate(ya2[0], yb2[0], e2)

    degi = jnp.concatenate([degi1, degi2])
    l2g = jnp.concatenate([l2g1, l2g2])
    cb2 = jnp.stack([c21_b, c22_b])
    return _final_stage(agg2_g1, agg2_g2, degi, l2g, cb2, pred_W.T, pred_b,
                        mb=1400)


# speedup vs baseline: 1.0223x; 1.0223x over previous
import functools

import jax
import jax.numpy as jnp
from jax import lax
from jax.experimental import pallas as pl
from jax.experimental.pallas import tpu as pltpu
from jax.experimental.pallas import tpu_sc as plsc

N = 35000
T = 35
D = 128
H = 32
HG = 64
OUT = 2
E = 1120000
G = N // T

NTILE = 16
NP = 36864
STRIPE = NP // NTILE
EW = 125
ROWS_PER_TILE = E // EW // NTILE
OBLK = 8
NOUT = ROWS_PER_TILE // OBLK

@functools.cache
def _sc_mesh():
    return plsc.VectorSubcoreMesh(core_axis_name="c", subcore_axis_name="s")


TCH = 7
NCH = T // TCH


def _scan_body(x_ref, wih_ref, whh_ref, b_ref, o_ref, h_sc, c_sc):
    s = pl.program_id(0)
    j = pl.program_id(1)
    is_bwd = s % 2
    wih = wih_ref[0]
    whh = whh_ref[0]
    b = b_ref[0]

    @pl.when(j == 0)
    def _():
        h_sc[...] = jnp.zeros((G, H), jnp.float32)
        c_sc[...] = jnp.zeros((G, H), jnp.float32)

    for tt in range(TCH):
        te = jnp.where(is_bwd == 1, TCH - 1 - tt, tt)
        h = h_sc[...]
        g = (jnp.dot(x_ref[0, te], wih, preferred_element_type=jnp.float32)
             + jnp.dot(h, whh, preferred_element_type=jnp.float32) + b)
        sg = jax.nn.sigmoid(g[:, 0:3 * H])
        tg = jnp.tanh(g[:, 3 * H:4 * H])
        c = sg[:, H:2 * H] * c_sc[...] + sg[:, 0:H] * tg
        h = sg[:, 2 * H:3 * H] * jnp.tanh(c)
        h_sc[...] = h
        c_sc[...] = c
        o_ref[0, te] = h


def _chunk_map(i, j):
    return jnp.where(i % 2 == 1, NCH - 1 - j, j)


def _lstm_scan(x, wih, whh, b):
    s, din = wih.shape[0], wih.shape[1]
    return pl.pallas_call(
        _scan_body,
        grid=(s, NCH),
        in_specs=[
            pl.BlockSpec((1, TCH, G, din),
                         lambda i, j: (i // 2, _chunk_map(i, j), 0, 0)),
            pl.BlockSpec((1, din, 4 * H), lambda i, j: (i, 0, 0)),
            pl.BlockSpec((1, H, 4 * H), lambda i, j: (i, 0, 0)),
            pl.BlockSpec((1, 1, 4 * H), lambda i, j: (i, 0, 0)),
        ],
        out_specs=pl.BlockSpec((1, TCH, G, H),
                               lambda i, j: (i, _chunk_map(i, j), 0, 0)),
        out_shape=jax.ShapeDtypeStruct((s, T, G, H), jnp.float32),
        scratch_shapes=[
            pltpu.VMEM((G, H), jnp.float32),
            pltpu.VMEM((G, H), jnp.float32),
        ],
    )(x, wih, whh, b.reshape(s, 1, 4 * H))


def _pre_body(wf_ref, dego_ref, linwt_ref, linb_ref, cw_ref, l2g_ref, xa_ref,
              xb_ref):
    wf = wf_ref[0]
    s_out = lax.rsqrt(jnp.maximum(dego_ref[0], 1.0))
    l2g_ref[0] = jnp.maximum(
        jnp.dot(wf, linwt_ref[...], preferred_element_type=jnp.float32)
        + linb_ref[...], 0.0)
    xw = jnp.dot(wf * s_out, cw_ref[0], preferred_element_type=jnp.float32)
    xa_ref[0] = xw[:, 0:HG // 2]
    xb_ref[0] = xw[:, HG // 2:HG]


def _pre_stage(wf, dego, lin_WT, lin_b, cw1, bm):
    return pl.pallas_call(
        _pre_body,
        grid=(1, N // bm),
        in_specs=[
            pl.BlockSpec((1, bm, HG), lambda i, j: (i, j, 0)),
            pl.BlockSpec((1, bm, 1), lambda i, j: (i, j, 0)),
            pl.BlockSpec((HG, HG), lambda i, j: (0, 0)),
            pl.BlockSpec((HG,), lambda i, j: (0,)),
            pl.BlockSpec((1, HG, HG), lambda i, j: (i, 0, 0)),
        ],
        out_specs=[
            pl.BlockSpec((1, bm, HG), lambda i, j: (i, j, 0)),
            pl.BlockSpec((1, bm, HG // 2), lambda i, j: (i, j, 0)),
            pl.BlockSpec((1, bm, HG // 2), lambda i, j: (i, j, 0)),
        ],
        out_shape=[
            jax.ShapeDtypeStruct((1, N, HG), jnp.float32),
            jax.ShapeDtypeStruct((1, N, HG // 2), jnp.float32),
            jax.ShapeDtypeStruct((1, N, HG // 2), jnp.float32),
        ],
    )(wf, dego, lin_WT, lin_b, cw1)


def _mid_body(aa_ref, ab_ref, degi_ref, dego_ref, l2g_ref, cb_ref, cw_ref,
              xa_ref, xb_ref):
    s_in = lax.rsqrt(jnp.maximum(degi_ref[0], 1.0))
    s_out = lax.rsqrt(jnp.maximum(dego_ref[0], 1.0))
    cb = cb_ref[0, 0]
    l2g = l2g_ref[0]
    Hh = HG // 2
    h1a = (jnp.maximum(aa_ref[0] * s_in + cb[0:Hh], 0.0) * 0.6
           + 0.4 * l2g[:, 0:Hh]) * s_out
    h1b = (jnp.maximum(ab_ref[0] * s_in + cb[Hh:HG], 0.0) * 0.6
           + 0.4 * l2g[:, Hh:HG]) * s_out
    cw = cw_ref[0]
    xw = (jnp.dot(h1a, cw[0:Hh], preferred_element_type=jnp.float32)
          + jnp.dot(h1b, cw[Hh:HG], preferred_element_type=jnp.float32))
    xa_ref[0] = xw[:, 0:Hh]
    xb_ref[0] = xw[:, Hh:HG]


def _mid_stage(agg, degi, dego, l2g, cb1, cw2, bm):
    Hh = HG // 2
    return pl.pallas_call(
        _mid_body,
        grid=(1, N // bm),
        in_specs=[
            pl.BlockSpec((1, bm, Hh), lambda i, j: (2 * i, j, 0)),
            pl.BlockSpec((1, bm, Hh), lambda i, j: (2 * i + 1, j, 0)),
            pl.BlockSpec((1, bm, 1), lambda i, j: (i, j, 0)),
            pl.BlockSpec((1, bm, 1), lambda i, j: (i, j, 0)),
            pl.BlockSpec((1, bm, HG), lambda i, j: (i, j, 0)),
            pl.BlockSpec((1, 1, HG), lambda i, j: (i, 0, 0)),
            pl.BlockSpec((1, HG, HG), lambda i, j: (i, 0, 0)),
        ],
        out_specs=[
            pl.BlockSpec((1, bm, Hh), lambda i, j: (i, j, 0)),
            pl.BlockSpec((1, bm, Hh), lambda i, j: (i, j, 0)),
        ],
        out_shape=[
            jax.ShapeDtypeStruct((1, N, Hh), jnp.float32),
            jax.ShapeDtypeStruct((1, N, Hh), jnp.float32),
        ],
    )(agg, agg, degi, dego, l2g, cb1.reshape(1, 1, HG), cw2)


def _final_body(a1_ref, a2_ref, degi_ref, l2g_ref, cb_ref, predwt_ref,
                predb_ref, o_ref, *, mb):
    gb = mb // T
    r = lax.broadcasted_iota(jnp.int32, (gb, mb), 0)
    cdiv = lax.broadcasted_iota(jnp.int32, (gb, mb), 1) // T
    P = jnp.where(r == cdiv, 1.0 / T, 0.0).astype(jnp.float32)
    Hh = HG // 2
    reps = []
    for s, ag_ref in ((0, a1_ref), (1, a2_ref)):
        s_in = lax.rsqrt(jnp.maximum(degi_ref[s], 1.0))
        cb = cb_ref[s]
        l2g = l2g_ref[s]
        h2a = (jnp.maximum(ag_ref[0] * s_in + cb[0:Hh], 0.0) * 0.6
               + 0.4 * l2g[:, 0:Hh])
        h2b = (jnp.maximum(ag_ref[1] * s_in + cb[Hh:HG], 0.0) * 0.6
               + 0.4 * l2g[:, Hh:HG])
        ra = jnp.dot(P, h2a, preferred_element_type=jnp.float32)
        rb = jnp.dot(P, h2b, preferred_element_type=jnp.float32)
        reps.append((ra, rb))
    da = jnp.tanh(reps[0][0] - reps[1][0])
    db = jnp.tanh(reps[0][1] - reps[1][1])
    pw = predwt_ref[...]
    o_ref[...] = (jnp.dot(da, pw[0:Hh], preferred_element_type=jnp.float32)
                  + jnp.dot(db, pw[Hh:HG], preferred_element_type=jnp.float32)
                  + predb_ref[...])


def _final_stage(a1, a2, degi, l2g, cb2, pred_WT, pred_b, mb):
    Hh = HG // 2
    return pl.pallas_call(
        functools.partial(_final_body, mb=mb),
        grid=(N // mb,),
        in_specs=[
            pl.BlockSpec((2, mb, Hh), lambda j: (0, j, 0)),
            pl.BlockSpec((2, mb, Hh), lambda j: (0, j, 0)),
            pl.BlockSpec((2, mb, 1), lambda j: (0, j, 0)),
            pl.BlockSpec((2, mb, HG), lambda j: (0, j, 0)),
            pl.BlockSpec((2, HG), lambda j: (0, 0)),
            pl.BlockSpec((HG, OUT), lambda j: (0, 0)),
            pl.BlockSpec((OUT,), lambda j: (0,)),
        ],
        out_specs=pl.BlockSpec((mb // T, OUT), lambda j: (j, 0)),
        out_shape=jax.ShapeDtypeStruct((G, OUT), jnp.float32),
    )(a1, a2, degi, l2g, cb2, pred_WT, pred_b)


def _deg_body(e1_ref, e2_ref, deg_ref, hs_ref, hd_ref, idx_ref, ones_ref,
              zb_ref, dsem):
    c = lax.axis_index("c")
    t = lax.axis_index("s")

    def fill(i, _):
        zb_ref[pl.ds(i * 16, 16)] = jnp.zeros((16,), jnp.float32)
        return 0

    lax.fori_loop(0, STRIPE // 16, fill, 0)
    for i in range(8):
        ones_ref[pl.ds(i * 16, 16)] = jnp.full((16,), 1.0, jnp.float32)
    pltpu.sync_copy(zb_ref, hs_ref.at[pl.ds(t * STRIPE, STRIPE)])
    pltpu.sync_copy(zb_ref, hd_ref.at[pl.ds(t * STRIPE, STRIPE)])
    plsc.subcore_barrier()

    def run(e_ref, dsem):
        for a, hist in ((0, hs_ref), (1, hd_ref)):
            def outer(o, _):
                pltpu.sync_copy(e_ref.at[a, t, pl.ds(o * OBLK, OBLK)],
                                idx_ref)
                scats = [
                    pltpu.async_copy(ones_ref.at[pl.ds(0, EW)],
                                     hist.at[idx_ref.at[b]], dsem, add=True)
                    for b in range(OBLK)
                ]
                for s_ in scats:
                    s_.wait()
                return 0

            lax.fori_loop(0, NOUT, outer, 0)

    pl.when(c == 0)(lambda: run(e1_ref, dsem))
    pl.when(c == 1)(lambda: run(e2_ref, dsem))
    plsc.subcore_barrier()
    pltpu.sync_copy(hs_ref.at[pl.ds(t * STRIPE, STRIPE)],
                    deg_ref.at[2 * c, 0, pl.ds(t * STRIPE, STRIPE)])
    pltpu.sync_copy(hd_ref.at[pl.ds(t * STRIPE, STRIPE)],
                    deg_ref.at[2 * c + 1, 0, pl.ds(t * STRIPE, STRIPE)])


def _sc_degrees(e1, e2):
    return pl.kernel(
        _deg_body,
        mesh=_sc_mesh(),
        compiler_params=pltpu.CompilerParams(use_tc_tiling_on_sc=False),
        out_type=jax.ShapeDtypeStruct((4, 1, NP), jnp.float32),
        scratch_types=[
            pltpu.VMEM_SHARED((NP,), jnp.float32),
            pltpu.VMEM_SHARED((NP,), jnp.float32),
            pltpu.VMEM((OBLK, EW), jnp.int32),
            pltpu.VMEM((128,), jnp.float32),
            pltpu.VMEM((STRIPE,), jnp.float32),
            pltpu.SemaphoreType.DMA,
        ],
    )(e1, e2)


ZROWS = 144
AIBLK = 40
ANOUT = ROWS_PER_TILE // AIBLK
SUBW = 5


def _agg_body(xa_ref, xb_ref, e_ref, out_ref, acc_ref, isrc_ref, idst_ref,
              rows_ref, zb_ref, gsem, ssem0, ssem1):
    c = lax.axis_index("c")
    t = lax.axis_index("s")
    ssems = (ssem0, ssem1)

    def fill(i, _):
        zb_ref[i, 0:16] = jnp.zeros((16,), jnp.float32)
        zb_ref[i, 16:32] = jnp.zeros((16,), jnp.float32)
        return 0

    lax.fori_loop(0, ZROWS, fill, 0)

    def drain_set(s):
        for b in range(SUBW):
            pltpu.make_async_copy(rows_ref.at[s, b],
                                  acc_ref.at[idst_ref.at[0]],
                                  ssems[s]).wait()

    def run_phase(xw_ref):
        for z in range(STRIPE // ZROWS):
            pltpu.sync_copy(
                zb_ref, acc_ref.at[pl.ds(t * STRIPE + z * ZROWS, ZROWS)])
        plsc.subcore_barrier()

        def outer(oi, _):
            pltpu.sync_copy(e_ref.at[0, t, pl.ds(oi * AIBLK, AIBLK)],
                            isrc_ref)
            pltpu.sync_copy(e_ref.at[1, t, pl.ds(oi * AIBLK, AIBLK)],
                            idst_ref)

            def sub(kp, _):
                for s in range(2):
                    k = kp * 2 + s
                    pl.when(jnp.logical_or(oi > 0, kp > 0))(
                        lambda s=s: drain_set(s))
                    gs = [
                        pltpu.async_copy(
                            xw_ref.at[isrc_ref.at[k * SUBW + b]],
                            rows_ref.at[s, b], gsem)
                        for b in range(SUBW)
                    ]
                    for g_ in gs:
                        g_.wait()
                    for b in range(SUBW):
                        pltpu.async_copy(rows_ref.at[s, b],
                                         acc_ref.at[idst_ref.at[k * SUBW + b]],
                                         ssems[s], add=True)
                return 0

            lax.fori_loop(0, AIBLK // SUBW // 2, sub, 0)
            return 0

        lax.fori_loop(0, ANOUT, outer, 0)
        for s in range(2):
            drain_set(s)
        plsc.subcore_barrier()
        pltpu.sync_copy(acc_ref.at[pl.ds(t * STRIPE, STRIPE)],
                        out_ref.at[c, pl.ds(t * STRIPE, STRIPE)])

    pl.when(c == 0)(lambda: run_phase(xa_ref))
    pl.when(c == 1)(lambda: run_phase(xb_ref))


def _sc_aggregate(xa, xb, e):
    return pl.kernel(
        _agg_body,
        mesh=_sc_mesh(),
        compiler_params=pltpu.CompilerParams(use_tc_tiling_on_sc=False),
        out_type=jax.ShapeDtypeStruct((2, NP, HG // 2), jnp.float32),
        scratch_types=[
            pltpu.VMEM_SHARED((NP, HG // 2), jnp.float32),
            pltpu.VMEM((AIBLK, EW), jnp.int32),
            pltpu.VMEM((AIBLK, EW), jnp.int32),
            pltpu.VMEM((2, SUBW, EW, HG // 2), jnp.float32),
            pltpu.VMEM((ZROWS, HG // 2), jnp.float32),
            pltpu.SemaphoreType.DMA,
            pltpu.SemaphoreType.DMA,
            pltpu.SemaphoreType.DMA,
        ],
    )(xa, xb, e)


def kernel(x1, edge_index1, x2, edge_index2, rnn1, rnn2, lin_W, lin_b,
           c11_W, c11_b, c21_W, c21_b, c12_W, c12_b, c22_W, c22_b,
           pred_W, pred_b):
    f32 = jnp.float32

    def gate_perm(w):
        return jnp.concatenate(
            [w[..., 0:2 * H], w[..., 3 * H:4 * H], w[..., 2 * H:3 * H]], -1)

    def layer_w(rnns, layer):
        wt, bt, ut = [], [], []
        for rnn in rnns:
            for d in range(2):
                Wih, Whh, bih, bhh = rnn[layer][d]
                wt.append(gate_perm(Wih.T))
                bt.append(gate_perm(bih + bhh))
                ut.append(gate_perm(Whh.T))
        return jnp.stack(wt), jnp.stack(bt), jnp.stack(ut)

    w1t, b1, u1t = layer_w((rnn1, rnn2), 0)
    w2t, b2, u2t = layer_w((rnn1, rnn2), 1)

    e1 = edge_index1.reshape(2, NTILE, ROWS_PER_TILE, EW)
    e2 = edge_index2.reshape(2, NTILE, ROWS_PER_TILE, EW)

    degs = _sc_degrees(e1, e2)
    dego = degs[0::2, 0, :N, None]
    degi = degs[1::2, 0, :N, None]

    def bilstm(x, w1t_i, u1t_i, b1_i, w2t_i, u2t_i, b2_i):
        xt = x.reshape(G, T, D).transpose(1, 0, 2).reshape(1, T, G, D)
        hs1 = _lstm_scan(xt, w1t_i, u1t_i, b1_i)
        wf_t = jnp.concatenate([hs1[0], hs1[1]], -1)[None]
        hs2 = _lstm_scan(wf_t, w2t_i, u2t_i, b2_i)
        return (jnp.concatenate([hs2[0], hs2[1]], -1)
                .transpose(1, 0, 2).reshape(1, N, 2 * H))

    wf1 = bilstm(x1, w1t[0:2], u1t[0:2], b1[0:2], w2t[0:2], u2t[0:2], b2[0:2])
    wf2 = bilstm(x2, w1t[2:4], u1t[2:4], b1[2:4], w2t[2:4], u2t[2:4], b2[2:4])

    dego1, degi1 = degs[0, 0, :N, None][None], degs[1, 0, :N, None][None]
    dego2, degi2 = degs[2, 0, :N, None][None], degs[3, 0, :N, None][None]

    l2g1, xa1, xb1 = _pre_stage(wf1, dego1, lin_W.T, lin_b, c11_W[None],
                                bm=5000)
    agg1_g1 = _sc_aggregate(xa1[0], xb1[0], e1)
    l2g2, xa2, xb2 = _pre_stage(wf2, dego2, lin_W.T, lin_b, c12_W[None],
                                bm=5000)
    agg1_g2 = _sc_aggregate(xa2[0], xb2[0], e2)
    ya1, yb1 = _mid_stage(agg1_g1, degi1, dego1, l2g1, c11_b[None],
                          c21_W[None], bm=5000)
    agg2_g1 = _sc_aggregate(ya1[0], yb1[0], e1)
    ya2, yb2 = _mid_stage(agg1_g2, degi2, dego2, l2g2, c12_b[None],
                          c22_W[None], bm=5000)
    agg2_g2 = _sc_aggregate(ya2[0], yb2[0], e2)

    degi = jnp.concatenate([degi1, degi2])
    l2g = jnp.concatenate([l2g1, l2g2])
    cb2 = jnp.stack([c21_b, c22_b])
    return _final_stage(agg2_g1, agg2_g2, degi, l2g, cb2, pred_W.T, pred_b,
                        mb=1400)
